# Initial kernel scaffold; baseline (speedup 1.0000x reference)
import jax
import jax.numpy as jnp
from jax.experimental import pallas as pl


def kernel(ego_feature, protocol_feature, Wv, bv, Woff, boff, Wa, ba, Wout, bout):
    raise NotImplementedError("write your pallas kernel here")


# speedup vs baseline: 250.5539x; 250.5539x over previous
import jax
import jax.numpy as jnp
from jax import lax
from jax.experimental import pallas as pl
from jax.experimental.pallas import tpu as pltpu
from jax.experimental.pallas import tpu_sc as plsc

_HEADS = 8
_POINTS = 4
_S = _HEADS * _POINTS
_CORNERS = 4
_K = _S * _CORNERS
_NC = 2
_NS = 16
_NW = _NC * _NS


def _pre_body(ego_ref, proto_ref, wv_ref, bv_ref, wox_ref, woy_ref,
              box_ref, boy_ref, wa_ref, ba_ref, gg_ref,
              vp_ref, idx_ref, wgt_ref, *, tn, h_img, w_img, n_tot):
    b = pl.program_id(0)
    nb = pl.program_id(1)
    eb = ego_ref[0]
    vb = proto_ref[0]
    dn = (((0,), (0,)), ((), ()))

    vp = lax.dot_general(vb, wv_ref[...], dn,
                         preferred_element_type=jnp.float32) + bv_ref[...]
    vp_ref[0] = vp

    offx = lax.dot_general(eb, wox_ref[...], dn,
                           preferred_element_type=jnp.float32) + box_ref[...]
    offy = lax.dot_general(eb, woy_ref[...], dn,
                           preferred_element_type=jnp.float32) + boy_ref[...]
    logit = lax.dot_general(eb, wa_ref[...], dn,
                            preferred_element_type=jnp.float32) + ba_ref[...]
    e = jnp.exp(logit)
    denom = lax.dot_general(e, gg_ref[...], (((1,), (0,)), ((), ())),
                            preferred_element_type=jnp.float32)
    aw = e / denom

    n = nb * tn + lax.broadcasted_iota(jnp.int32, (tn, 1), 0)
    m = n >> 6
    i = (m * 21846) >> 16
    j = n - i * w_img
    x = j.astype(jnp.float32) + offx
    y = i.astype(jnp.float32) + offy
    x0f = jnp.floor(x)
    y0f = jnp.floor(y)
    fx1 = x - x0f
    fx0 = 1.0 - fx1
    fy1 = y - y0f
    fy0 = 1.0 - fy1
    x0 = x0f.astype(jnp.int32)
    y0 = y0f.astype(jnp.int32)
    x1 = x0 + 1
    y1 = y0 + 1

    head = lax.broadcasted_iota(jnp.int32, (tn, _S), 1) >> 2
    base = b * n_tot

    def corner(xc, yc, wx, wy):
        valid = ((xc >= 0) & (xc <= w_img - 1) &
                 (yc >= 0) & (yc <= h_img - 1))
        w = wx * wy * aw * valid.astype(jnp.float32)
        xq = jnp.clip(xc, 0, w_img - 1)
        yq = jnp.clip(yc, 0, h_img - 1)
        r = ((base + yq * w_img + xq) << 3) + head
        return r, w

    r00, w00 = corner(x0, y0, fx0, fy0)
    r10, w10 = corner(x1, y0, fx1, fy0)
    r01, w01 = corner(x0, y1, fx0, fy1)
    r11, w11 = corner(x1, y1, fx1, fy1)
    idx_ref[0] = jnp.concatenate([r00, r10, r01, r11], axis=1)
    wgt_ref[0] = jnp.concatenate([w00, w10, w01, w11], axis=1)


def _post_body(samp_ref, ego_ref, wout_ref, bout_ref, out_ref):
    sb = samp_ref[0]
    o = lax.dot_general(wout_ref[...], sb, (((0,), (1,)), ((), ())),
                        preferred_element_type=jnp.float32)
    out_ref[0] = o + bout_ref[...] + ego_ref[0]


def _sc_gather(table, idxf, wgtf, *, bsn, dph, cq):
    qw = bsn // _NW
    nchunk = qw // cq
    mesh = plsc.VectorSubcoreMesh(core_axis_name="c", subcore_axis_name="s")

    def body(table_hbm, idx_hbm, wgt_hbm, out_hbm,
             idx_v, wgt_v, rows_v, out_v, gsem):
        wid = lax.axis_index("s") * _NC + lax.axis_index("c")
        base = wid * qw

        def chunk(g, carry):
            q0 = base + g * cq
            pltpu.sync_copy(idx_hbm.at[pl.ds(q0, cq)], idx_v)
            pltpu.sync_copy(wgt_hbm.at[pl.ds(q0, cq)], wgt_v)
            handles = []
            for q in range(cq):
                handles.append(pltpu.async_copy(
                    table_hbm.at[idx_v.at[q]],
                    rows_v.at[pl.ds(q * _K, _K)], gsem))
            for hnd in handles:
                hnd.wait()

            def per_query(q, c2):
                acc = [jnp.zeros((dph,), jnp.float32) for _ in range(_HEADS)]
                for j16 in range(_K // 16):
                    wv = wgt_v[q, pl.ds(j16 * 16, 16)]
                    for l in range(16):
                        j = j16 * 16 + l
                        r = rows_v[q * _K + j, :]
                        hh = (j % _S) >> 2
                        acc[hh] = acc[hh] + r * wv[l]
                for hh in range(_HEADS):
                    out_v[q, pl.ds(hh * dph, dph)] = acc[hh]
                return c2

            lax.fori_loop(0, cq, per_query, 0)
            pltpu.sync_copy(out_v, out_hbm.at[pl.ds(q0, cq)])
            return carry

        lax.fori_loop(0, nchunk, chunk, 0)

    f = pl.kernel(
        body,
        out_type=jax.ShapeDtypeStruct((bsn, _K), jnp.float32),
        mesh=mesh,
        compiler_params=pltpu.CompilerParams(use_tc_tiling_on_sc=False),
        scratch_types=[
            pltpu.VMEM((cq, _K), jnp.int32),
            pltpu.VMEM((cq, _K), jnp.float32),
            pltpu.VMEM((cq * _K, dph), jnp.float32),
            pltpu.VMEM((cq, _K), jnp.float32),
            pltpu.SemaphoreType.DMA,
        ],
    )
    return f(table, idxf, wgtf)


def kernel(ego_feature, protocol_feature, Wv, bv, Woff, boff, Wa, ba,
           Wout, bout):
    bs, C, H, W = ego_feature.shape
    N = H * W
    dph = C // _HEADS
    TN = 512
    CQ = 16

    ego3 = ego_feature.reshape(bs, C, N)
    proto3 = protocol_feature.reshape(bs, C, N)
    Woffx = Woff[:, 0::2]
    Woffy = Woff[:, 1::2]
    bo = boff.reshape(_S, 2)
    boffx = bo[:, 0].reshape(1, _S)
    boffy = bo[:, 1].reshape(1, _S)
    bv2 = bv.reshape(1, C)
    ba2 = ba.reshape(1, _S)
    bout2 = bout.reshape(C, 1)
    GG = jnp.kron(jnp.eye(_HEADS, dtype=jnp.float32),
                  jnp.ones((_POINTS, _POINTS), jnp.float32))

    nblk = N // TN
    grid = (bs, nblk)

    import functools
    pre = pl.pallas_call(
        functools.partial(_pre_body, tn=TN, h_img=H, w_img=W, n_tot=N),
        grid=grid,
        in_specs=[
            pl.BlockSpec((1, C, TN), lambda b, nb: (b, 0, nb)),
            pl.BlockSpec((1, C, TN), lambda b, nb: (b, 0, nb)),
            pl.BlockSpec((C, C), lambda b, nb: (0, 0)),
            pl.BlockSpec((1, C), lambda b, nb: (0, 0)),
            pl.BlockSpec((C, _S), lambda b, nb: (0, 0)),
            pl.BlockSpec((C, _S), lambda b, nb: (0, 0)),
            pl.BlockSpec((1, _S), lambda b, nb: (0, 0)),
            pl.BlockSpec((1, _S), lambda b, nb: (0, 0)),
            pl.BlockSpec((C, _S), lambda b, nb: (0, 0)),
            pl.BlockSpec((1, _S), lambda b, nb: (0, 0)),
            pl.BlockSpec((_S, _S), lambda b, nb: (0, 0)),
        ],
        out_specs=[
            pl.BlockSpec((1, TN, C), lambda b, nb: (b, nb, 0)),
            pl.BlockSpec((1, TN, _K), lambda b, nb: (b, nb, 0)),
            pl.BlockSpec((1, TN, _K), lambda b, nb: (b, nb, 0)),
        ],
        out_shape=[
            jax.ShapeDtypeStruct((bs, N, C), jnp.float32),
            jax.ShapeDtypeStruct((bs, N, _K), jnp.int32),
            jax.ShapeDtypeStruct((bs, N, _K), jnp.float32),
        ],
    )
    vp, idxa, wgta = pre(ego3, proto3, Wv, bv2, Woffx, Woffy,
                         boffx, boffy, Wa, ba2, GG)

    table = vp.reshape(bs * N * _HEADS, dph)
    idxf = idxa.reshape(bs * N, _K)
    wgtf = wgta.reshape(bs * N, _K)
    samp = _sc_gather(table, idxf, wgtf, bsn=bs * N, dph=dph, cq=CQ)

    post = pl.pallas_call(
        _post_body,
        grid=grid,
        in_specs=[
            pl.BlockSpec((1, TN, C), lambda b, nb: (b, nb, 0)),
            pl.BlockSpec((1, C, TN), lambda b, nb: (b, 0, nb)),
            pl.BlockSpec((C, C), lambda b, nb: (0, 0)),
            pl.BlockSpec((C, 1), lambda b, nb: (0, 0)),
        ],
        out_specs=pl.BlockSpec((1, C, TN), lambda b, nb: (b, 0, nb)),
        out_shape=jax.ShapeDtypeStruct((bs, C, N), jnp.float32),
    )
    out3 = post(samp.reshape(bs, N, C), ego3, Wout, bout2)
    return out3.reshape(bs, C, H, W)


# speedup vs baseline: 359.9920x; 1.4368x over previous
import jax
import jax.numpy as jnp
from jax import lax
from jax.experimental import pallas as pl
from jax.experimental.pallas import tpu as pltpu
from jax.experimental.pallas import tpu_sc as plsc

_HEADS = 8
_POINTS = 4
_S = _HEADS * _POINTS
_CORNERS = 4
_K = _S * _CORNERS
_NC = 2
_NS = 16
_NW = _NC * _NS


def _pre_body(ego_ref, proto_ref, wv_ref, bv_ref, wox_ref, woy_ref,
              box_ref, boy_ref, wa_ref, ba_ref, gg_ref,
              vp_ref, idx_ref, wgt_ref, *, tn, h_img, w_img, n_tot):
    b = pl.program_id(0)
    nb = pl.program_id(1)
    eb = ego_ref[0]
    vb = proto_ref[0]
    dn = (((0,), (0,)), ((), ()))

    vp = lax.dot_general(vb, wv_ref[...], dn,
                         preferred_element_type=jnp.float32) + bv_ref[...]
    vp_ref[0] = vp

    offx = lax.dot_general(eb, wox_ref[...], dn,
                           preferred_element_type=jnp.float32) + box_ref[...]
    offy = lax.dot_general(eb, woy_ref[...], dn,
                           preferred_element_type=jnp.float32) + boy_ref[...]
    logit = lax.dot_general(eb, wa_ref[...], dn,
                            preferred_element_type=jnp.float32) + ba_ref[...]
    e = jnp.exp(logit)
    denom = lax.dot_general(e, gg_ref[...], (((1,), (0,)), ((), ())),
                            preferred_element_type=jnp.float32)
    aw = e / denom

    n = nb * tn + lax.broadcasted_iota(jnp.int32, (tn, 1), 0)
    m = n >> 6
    i = (m * 21846) >> 16
    j = n - i * w_img
    x = j.astype(jnp.float32) + offx
    y = i.astype(jnp.float32) + offy
    x0f = jnp.floor(x)
    y0f = jnp.floor(y)
    fx1 = x - x0f
    fx0 = 1.0 - fx1
    fy1 = y - y0f
    fy0 = 1.0 - fy1
    x0 = x0f.astype(jnp.int32)
    y0 = y0f.astype(jnp.int32)
    x1 = x0 + 1
    y1 = y0 + 1

    head = lax.broadcasted_iota(jnp.int32, (tn, _S), 1) >> 2
    base = b * n_tot

    def corner(xc, yc, wx, wy):
        valid = ((xc >= 0) & (xc <= w_img - 1) &
                 (yc >= 0) & (yc <= h_img - 1))
        w = wx * wy * aw * valid.astype(jnp.float32)
        xq = jnp.clip(xc, 0, w_img - 1)
        yq = jnp.clip(yc, 0, h_img - 1)
        r = ((base + yq * w_img + xq) << 3) + head
        return r, w

    r00, w00 = corner(x0, y0, fx0, fy0)
    r10, w10 = corner(x1, y0, fx1, fy0)
    r01, w01 = corner(x0, y1, fx0, fy1)
    r11, w11 = corner(x1, y1, fx1, fy1)
    idx_ref[0] = jnp.concatenate([r00, r10, r01, r11], axis=1)
    wgt_ref[0] = jnp.concatenate([w00, w10, w01, w11], axis=1)


def _post_body(samp_ref, ego_ref, wout_ref, bout_ref, out_ref):
    sb = samp_ref[0]
    o = lax.dot_general(wout_ref[...], sb, (((0,), (1,)), ((), ())),
                        preferred_element_type=jnp.float32)
    out_ref[0] = o + bout_ref[...] + ego_ref[0]


def _sc_gather(table, idxf, wgtf, *, bsn, dph, cq):
    qw = bsn // _NW
    nchunk = qw // cq
    mesh = plsc.VectorSubcoreMesh(core_axis_name="c", subcore_axis_name="s")

    def body(table_hbm, idx_hbm, wgt_hbm, out_hbm,
             idx_v, wgt_v, rows_v, out_v, gsem, iwsem, osem):
        wid = lax.axis_index("s") * _NC + lax.axis_index("c")
        base = wid * qw

        def start_iw(g, slot):
            q0 = base + g * cq
            pltpu.async_copy(idx_hbm.at[pl.ds(q0, cq)],
                             idx_v.at[pl.ds(slot * cq, cq)], iwsem)
            pltpu.async_copy(wgt_hbm.at[pl.ds(q0, cq)],
                             wgt_v.at[pl.ds(slot * cq, cq)], iwsem)

        def wait_iw():
            for _ in range(2):
                pltpu.make_async_copy(
                    idx_hbm.at[pl.ds(base, cq)],
                    idx_v.at[pl.ds(0, cq)], iwsem).wait()

        def start_gathers(slot):
            for q in range(cq):
                pltpu.async_copy(
                    table_hbm.at[idx_v.at[slot * cq + q]],
                    rows_v.at[pl.ds((slot * cq + q) * _K, _K)], gsem)

        def wait_gathers():
            for _ in range(cq):
                pltpu.make_async_copy(
                    table_hbm.at[idx_v.at[0]],
                    rows_v.at[pl.ds(0, _K)], gsem).wait()

        def wait_out():
            pltpu.make_async_copy(
                out_v.at[pl.ds(0, cq)],
                out_hbm.at[pl.ds(base, cq)], osem).wait()

        pltpu.sync_copy(idx_hbm.at[pl.ds(base, cq)], idx_v.at[pl.ds(0, cq)])
        pltpu.sync_copy(wgt_hbm.at[pl.ds(base, cq)], wgt_v.at[pl.ds(0, cq)])
        start_gathers(0)
        start_iw(1, 1)

        def chunk(g, carry):
            slot = g & 1
            wait_gathers()

            @pl.when(g < nchunk - 1)
            def _():
                wait_iw()
                start_gathers(1 - slot)

            @pl.when(g >= 2)
            def _():
                wait_out()

            def per_query(q, c2):
                acc = [jnp.zeros((dph,), jnp.float32) for _ in range(_HEADS)]
                for j16 in range(_K // 16):
                    wv = wgt_v[slot * cq + q, pl.ds(j16 * 16, 16)]
                    for l in range(16):
                        j = j16 * 16 + l
                        r = rows_v[(slot * cq + q) * _K + j, :]
                        hh = (j % _S) >> 2
                        acc[hh] = acc[hh] + r * wv[l]
                for hh in range(_HEADS):
                    out_v[slot * cq + q, pl.ds(hh * dph, dph)] = acc[hh]
                return c2

            lax.fori_loop(0, cq, per_query, 0)
            pltpu.async_copy(out_v.at[pl.ds(slot * cq, cq)],
                             out_hbm.at[pl.ds(base + g * cq, cq)], osem)

            @pl.when(g < nchunk - 2)
            def _():
                start_iw(g + 2, slot)

            return carry

        lax.fori_loop(0, nchunk, chunk, 0)
        wait_out()
        wait_out()

    f = pl.kernel(
        body,
        out_type=jax.ShapeDtypeStruct((bsn, _K), jnp.float32),
        mesh=mesh,
        compiler_params=pltpu.CompilerParams(use_tc_tiling_on_sc=False),
        scratch_types=[
            pltpu.VMEM((2 * cq, _K), jnp.int32),
            pltpu.VMEM((2 * cq, _K), jnp.float32),
            pltpu.VMEM((2 * cq * _K, dph), jnp.float32),
            pltpu.VMEM((2 * cq, _K), jnp.float32),
            pltpu.SemaphoreType.DMA,
            pltpu.SemaphoreType.DMA,
            pltpu.SemaphoreType.DMA,
        ],
    )
    return f(table, idxf, wgtf)


def kernel(ego_feature, protocol_feature, Wv, bv, Woff, boff, Wa, ba,
           Wout, bout):
    bs, C, H, W = ego_feature.shape
    N = H * W
    dph = C // _HEADS
    TN = 512
    CQ = 16

    ego3 = ego_feature.reshape(bs, C, N)
    proto3 = protocol_feature.reshape(bs, C, N)
    Woffx = Woff[:, 0::2]
    Woffy = Woff[:, 1::2]
    bo = boff.reshape(_S, 2)
    boffx = bo[:, 0].reshape(1, _S)
    boffy = bo[:, 1].reshape(1, _S)
    bv2 = bv.reshape(1, C)
    ba2 = ba.reshape(1, _S)
    bout2 = bout.reshape(C, 1)
    GG = jnp.kron(jnp.eye(_HEADS, dtype=jnp.float32),
                  jnp.ones((_POINTS, _POINTS), jnp.float32))

    nblk = N // TN
    grid = (bs, nblk)

    import functools
    pre = pl.pallas_call(
        functools.partial(_pre_body, tn=TN, h_img=H, w_img=W, n_tot=N),
        grid=grid,
        in_specs=[
            pl.BlockSpec((1, C, TN), lambda b, nb: (b, 0, nb)),
            pl.BlockSpec((1, C, TN), lambda b, nb: (b, 0, nb)),
            pl.BlockSpec((C, C), lambda b, nb: (0, 0)),
            pl.BlockSpec((1, C), lambda b, nb: (0, 0)),
            pl.BlockSpec((C, _S), lambda b, nb: (0, 0)),
            pl.BlockSpec((C, _S), lambda b, nb: (0, 0)),
            pl.BlockSpec((1, _S), lambda b, nb: (0, 0)),
            pl.BlockSpec((1, _S), lambda b, nb: (0, 0)),
            pl.BlockSpec((C, _S), lambda b, nb: (0, 0)),
            pl.BlockSpec((1, _S), lambda b, nb: (0, 0)),
            pl.BlockSpec((_S, _S), lambda b, nb: (0, 0)),
        ],
        out_specs=[
            pl.BlockSpec((1, TN, C), lambda b, nb: (b, nb, 0)),
            pl.BlockSpec((1, TN, _K), lambda b, nb: (b, nb, 0)),
            pl.BlockSpec((1, TN, _K), lambda b, nb: (b, nb, 0)),
        ],
        out_shape=[
            jax.ShapeDtypeStruct((bs, N, C), jnp.float32),
            jax.ShapeDtypeStruct((bs, N, _K), jnp.int32),
            jax.ShapeDtypeStruct((bs, N, _K), jnp.float32),
        ],
    )
    vp, idxa, wgta = pre(ego3, proto3, Wv, bv2, Woffx, Woffy,
                         boffx, boffy, Wa, ba2, GG)

    table = vp.reshape(bs * N * _HEADS, dph)
    idxf = idxa.reshape(bs * N, _K)
    wgtf = wgta.reshape(bs * N, _K)
    samp = _sc_gather(table, idxf, wgtf, bsn=bs * N, dph=dph, cq=CQ)

    post = pl.pallas_call(
        _post_body,
        grid=grid,
        in_specs=[
            pl.BlockSpec((1, TN, C), lambda b, nb: (b, nb, 0)),
            pl.BlockSpec((1, C, TN), lambda b, nb: (b, 0, nb)),
            pl.BlockSpec((C, C), lambda b, nb: (0, 0)),
            pl.BlockSpec((C, 1), lambda b, nb: (0, 0)),
        ],
        out_specs=pl.BlockSpec((1, C, TN), lambda b, nb: (b, 0, nb)),
        out_shape=jax.ShapeDtypeStruct((bs, C, N), jnp.float32),
    )
    out3 = post(samp.reshape(bs, N, C), ego3, Wout, bout2)
    return out3.reshape(bs, C, H, W)


# speedup vs baseline: 375.1614x; 1.0421x over previous
import jax
import jax.numpy as jnp
from jax import lax
from jax.experimental import pallas as pl
from jax.experimental.pallas import tpu as pltpu
from jax.experimental.pallas import tpu_sc as plsc

_HEADS = 8
_POINTS = 4
_S = _HEADS * _POINTS
_CORNERS = 4
_K = _S * _CORNERS
_NC = 2
_NS = 16
_NW = _NC * _NS


def _pre_body(ego_ref, proto_ref, wv_ref, bv_ref, wox_ref, woy_ref,
              box_ref, boy_ref, wa_ref, ba_ref, gg_ref,
              vp_ref, idx_ref, wgt_ref, *, tn, h_img, w_img, n_tot):
    b = pl.program_id(0)
    nb = pl.program_id(1)
    eb = ego_ref[0]
    vb = proto_ref[0]
    dn = (((0,), (0,)), ((), ()))

    vp = lax.dot_general(vb, wv_ref[...], dn,
                         preferred_element_type=jnp.float32) + bv_ref[...]
    vp_ref[0] = vp

    offx = lax.dot_general(eb, wox_ref[...], dn,
                           preferred_element_type=jnp.float32) + box_ref[...]
    offy = lax.dot_general(eb, woy_ref[...], dn,
                           preferred_element_type=jnp.float32) + boy_ref[...]
    logit = lax.dot_general(eb, wa_ref[...], dn,
                            preferred_element_type=jnp.float32) + ba_ref[...]
    e = jnp.exp(logit)
    denom = lax.dot_general(e, gg_ref[...], (((1,), (0,)), ((), ())),
                            preferred_element_type=jnp.float32)
    aw = e / denom

    lane = lax.broadcasted_iota(jnp.int32, (tn, _K), 1)
    cor = lane >> 5
    dx = (cor & 1).astype(jnp.float32)
    dy = (cor >> 1).astype(jnp.float32)
    head = (lane & (_S - 1)) >> 2

    n = nb * tn + lax.broadcasted_iota(jnp.int32, (tn, 1), 0)
    m = n >> 6
    i = (m * 21846) >> 16
    j = n - i * w_img
    x = j.astype(jnp.float32) + offx
    y = i.astype(jnp.float32) + offy
    x0f = jnp.floor(x)
    y0f = jnp.floor(y)
    fx1 = x - x0f
    fy1 = y - y0f
    xcf = x0f + dx
    ycf = y0f + dy
    wx = dx * fx1 + (1.0 - dx) * (1.0 - fx1)
    wy = dy * fy1 + (1.0 - dy) * (1.0 - fy1)
    valid = ((xcf >= 0.0) & (xcf <= w_img - 1) &
             (ycf >= 0.0) & (ycf <= h_img - 1))
    w = wx * wy * aw * valid.astype(jnp.float32)
    xi = jnp.clip(xcf, 0.0, w_img - 1).astype(jnp.int32)
    yi = jnp.clip(ycf, 0.0, h_img - 1).astype(jnp.int32)
    base = b * n_tot
    idx_ref[0] = ((base + yi * w_img + xi) << 3) + head
    wgt_ref[0] = w


def _post_body(samp_ref, ego_ref, wout_ref, bout_ref, out_ref):
    sb = samp_ref[0]
    o = lax.dot_general(wout_ref[...], sb, (((0,), (1,)), ((), ())),
                        preferred_element_type=jnp.float32)
    out_ref[0] = o + bout_ref[...] + ego_ref[0]


def _sc_gather(table, idxf, wgtf, *, bsn, dph, cq):
    qw = bsn // _NW
    nchunk = qw // cq
    mesh = plsc.VectorSubcoreMesh(core_axis_name="c", subcore_axis_name="s")

    def body(table_hbm, idx_hbm, wgt_hbm, out_hbm,
             idx_v, wgt_v, rows_v, out_v, gsem, iwsem, osem):
        wid = lax.axis_index("s") * _NC + lax.axis_index("c")
        base = wid * qw

        def start_iw(g, slot):
            q0 = base + g * cq
            pltpu.async_copy(idx_hbm.at[pl.ds(q0, cq)],
                             idx_v.at[pl.ds(slot * cq, cq)], iwsem)
            pltpu.async_copy(wgt_hbm.at[pl.ds(q0, cq)],
                             wgt_v.at[pl.ds(slot * cq, cq)], iwsem)

        def wait_iw():
            for _ in range(2):
                pltpu.make_async_copy(
                    idx_hbm.at[pl.ds(base, cq)],
                    idx_v.at[pl.ds(0, cq)], iwsem).wait()

        def start_gathers(slot):
            for q in range(cq):
                pltpu.async_copy(
                    table_hbm.at[idx_v.at[slot * cq + q]],
                    rows_v.at[pl.ds((slot * cq + q) * _K, _K)], gsem)

        def wait_gathers():
            for _ in range(cq):
                pltpu.make_async_copy(
                    table_hbm.at[idx_v.at[0]],
                    rows_v.at[pl.ds(0, _K)], gsem).wait()

        def wait_out():
            pltpu.make_async_copy(
                out_v.at[pl.ds(0, cq)],
                out_hbm.at[pl.ds(base, cq)], osem).wait()

        pltpu.sync_copy(idx_hbm.at[pl.ds(base, cq)], idx_v.at[pl.ds(0, cq)])
        pltpu.sync_copy(wgt_hbm.at[pl.ds(base, cq)], wgt_v.at[pl.ds(0, cq)])
        start_gathers(0)
        start_iw(1, 1)

        def chunk(g, carry):
            slot = g & 1
            wait_gathers()

            @pl.when(g < nchunk - 1)
            def _():
                wait_iw()
                start_gathers(1 - slot)

            @pl.when(g >= 2)
            def _():
                wait_out()

            def per_query(q, c2):
                acc = [jnp.zeros((dph,), jnp.float32) for _ in range(_HEADS)]
                for j16 in range(_K // 16):
                    wv = wgt_v[slot * cq + q, pl.ds(j16 * 16, 16)]
                    for l in range(16):
                        j = j16 * 16 + l
                        r = rows_v[(slot * cq + q) * _K + j, :]
                        hh = (j % _S) >> 2
                        acc[hh] = acc[hh] + r * wv[l]
                for hh in range(_HEADS):
                    out_v[slot * cq + q, pl.ds(hh * dph, dph)] = acc[hh]
                return c2

            lax.fori_loop(0, cq, per_query, 0)
            pltpu.async_copy(out_v.at[pl.ds(slot * cq, cq)],
                             out_hbm.at[pl.ds(base + g * cq, cq)], osem)

            @pl.when(g < nchunk - 2)
            def _():
                start_iw(g + 2, slot)

            return carry

        lax.fori_loop(0, nchunk, chunk, 0)
        wait_out()
        wait_out()

    f = pl.kernel(
        body,
        out_type=jax.ShapeDtypeStruct((bsn, _K), jnp.float32),
        mesh=mesh,
        compiler_params=pltpu.CompilerParams(use_tc_tiling_on_sc=False),
        scratch_types=[
            pltpu.VMEM((2 * cq, _K), jnp.int32),
            pltpu.VMEM((2 * cq, _K), jnp.float32),
            pltpu.VMEM((2 * cq * _K, dph), jnp.float32),
            pltpu.VMEM((2 * cq, _K), jnp.float32),
            pltpu.SemaphoreType.DMA,
            pltpu.SemaphoreType.DMA,
            pltpu.SemaphoreType.DMA,
        ],
    )
    return f(table, idxf, wgtf)


def kernel(ego_feature, protocol_feature, Wv, bv, Woff, boff, Wa, ba,
           Wout, bout):
    bs, C, H, W = ego_feature.shape
    N = H * W
    dph = C // _HEADS
    TN = 512
    CQ = 16

    ego3 = ego_feature.reshape(bs, C, N)
    proto3 = protocol_feature.reshape(bs, C, N)
    Woffx = jnp.concatenate([Woff[:, 0::2]] * _CORNERS, axis=1)
    Woffy = jnp.concatenate([Woff[:, 1::2]] * _CORNERS, axis=1)
    bo = boff.reshape(_S, 2)
    boffx = jnp.concatenate([bo[:, 0].reshape(1, _S)] * _CORNERS, axis=1)
    boffy = jnp.concatenate([bo[:, 1].reshape(1, _S)] * _CORNERS, axis=1)
    Wa4 = jnp.concatenate([Wa] * _CORNERS, axis=1)
    ba4 = jnp.concatenate([ba.reshape(1, _S)] * _CORNERS, axis=1)
    bv2 = bv.reshape(1, C)
    bout2 = bout.reshape(C, 1)
    GG = jnp.kron(jnp.eye(_S, dtype=jnp.float32),
                  jnp.ones((_POINTS, _POINTS), jnp.float32))

    nblk = N // TN
    grid = (bs, nblk)

    import functools
    pre = pl.pallas_call(
        functools.partial(_pre_body, tn=TN, h_img=H, w_img=W, n_tot=N),
        grid=grid,
        in_specs=[
            pl.BlockSpec((1, C, TN), lambda b, nb: (b, 0, nb)),
            pl.BlockSpec((1, C, TN), lambda b, nb: (b, 0, nb)),
            pl.BlockSpec((C, C), lambda b, nb: (0, 0)),
            pl.BlockSpec((1, C), lambda b, nb: (0, 0)),
            pl.BlockSpec((C, _K), lambda b, nb: (0, 0)),
            pl.BlockSpec((C, _K), lambda b, nb: (0, 0)),
            pl.BlockSpec((1, _K), lambda b, nb: (0, 0)),
            pl.BlockSpec((1, _K), lambda b, nb: (0, 0)),
            pl.BlockSpec((C, _K), lambda b, nb: (0, 0)),
            pl.BlockSpec((1, _K), lambda b, nb: (0, 0)),
            pl.BlockSpec((_K, _K), lambda b, nb: (0, 0)),
        ],
        out_specs=[
            pl.BlockSpec((1, TN, C), lambda b, nb: (b, nb, 0)),
            pl.BlockSpec((1, TN, _K), lambda b, nb: (b, nb, 0)),
            pl.BlockSpec((1, TN, _K), lambda b, nb: (b, nb, 0)),
        ],
        out_shape=[
            jax.ShapeDtypeStruct((bs, N, C), jnp.float32),
            jax.ShapeDtypeStruct((bs, N, _K), jnp.int32),
            jax.ShapeDtypeStruct((bs, N, _K), jnp.float32),
        ],
    )
    vp, idxa, wgta = pre(ego3, proto3, Wv, bv2, Woffx, Woffy,
                         boffx, boffy, Wa4, ba4, GG)

    table = vp.reshape(bs * N * _HEADS, dph)
    idxf = idxa.reshape(bs * N, _K)
    wgtf = wgta.reshape(bs * N, _K)
    samp = _sc_gather(table, idxf, wgtf, bsn=bs * N, dph=dph, cq=CQ)

    post = pl.pallas_call(
        _post_body,
        grid=grid,
        in_specs=[
            pl.BlockSpec((1, TN, C), lambda b, nb: (b, nb, 0)),
            pl.BlockSpec((1, C, TN), lambda b, nb: (b, 0, nb)),
            pl.BlockSpec((C, C), lambda b, nb: (0, 0)),
            pl.BlockSpec((C, 1), lambda b, nb: (0, 0)),
        ],
        out_specs=pl.BlockSpec((1, C, TN), lambda b, nb: (b, 0, nb)),
        out_shape=jax.ShapeDtypeStruct((bs, C, N), jnp.float32),
    )
    out3 = post(samp.reshape(bs, N, C), ego3, Wout, bout2)
    return out3.reshape(bs, C, H, W)


# speedup vs baseline: 397.3733x; 1.0592x over previous
import jax
import jax.numpy as jnp
from jax import lax
from jax.experimental import pallas as pl
from jax.experimental.pallas import tpu as pltpu
from jax.experimental.pallas import tpu_sc as plsc

_HEADS = 8
_POINTS = 4
_S = _HEADS * _POINTS
_CORNERS = 4
_K = _S * _CORNERS
_NC = 2
_NS = 16
_NW = _NC * _NS


def _pre_body(ego_ref, proto_ref, wv_ref, bv_ref, wox_ref, woy_ref,
              box_ref, boy_ref, wa_ref, ba_ref, gg_ref,
              vp_ref, idx_ref, wgt_ref, *, tn, h_img, w_img, n_tot):
    b = pl.program_id(0)
    nb = pl.program_id(1)
    eb = ego_ref[0]
    vb = proto_ref[0]
    dn = (((0,), (0,)), ((), ()))

    vp = lax.dot_general(vb, wv_ref[...], dn,
                         preferred_element_type=jnp.float32) + bv_ref[...]
    vp_ref[0] = vp

    offx = lax.dot_general(eb, wox_ref[...], dn,
                           preferred_element_type=jnp.float32) + box_ref[...]
    offy = lax.dot_general(eb, woy_ref[...], dn,
                           preferred_element_type=jnp.float32) + boy_ref[...]
    logit = lax.dot_general(eb, wa_ref[...], dn,
                            preferred_element_type=jnp.float32) + ba_ref[...]
    e = jnp.exp(logit)
    denom = lax.dot_general(e, gg_ref[...], (((1,), (0,)), ((), ())),
                            preferred_element_type=jnp.float32)
    aw = e / denom

    lane = lax.broadcasted_iota(jnp.int32, (tn, _K), 1)
    cor = lane >> 5
    dx = (cor & 1).astype(jnp.float32)
    dy = (cor >> 1).astype(jnp.float32)
    head = (lane & (_S - 1)) >> 2

    n = nb * tn + lax.broadcasted_iota(jnp.int32, (tn, 1), 0)
    m = n >> 6
    i = (m * 21846) >> 16
    j = n - i * w_img
    x = j.astype(jnp.float32) + offx
    y = i.astype(jnp.float32) + offy
    x0f = jnp.floor(x)
    y0f = jnp.floor(y)
    fx1 = x - x0f
    fy1 = y - y0f
    xcf = x0f + dx
    ycf = y0f + dy
    wx = dx * fx1 + (1.0 - dx) * (1.0 - fx1)
    wy = dy * fy1 + (1.0 - dy) * (1.0 - fy1)
    valid = ((xcf >= 0.0) & (xcf <= w_img - 1) &
             (ycf >= 0.0) & (ycf <= h_img - 1))
    w = wx * wy * aw * valid.astype(jnp.float32)
    xi = jnp.clip(xcf, 0.0, w_img - 1).astype(jnp.int32)
    yi = jnp.clip(ycf, 0.0, h_img - 1).astype(jnp.int32)
    base = b * n_tot
    idx_ref[0] = ((base + yi * w_img + xi) << 3) + head
    wgt_ref[0] = w


def _post_body(samp_ref, ego_ref, wout_ref, bout_ref, out_ref):
    sb = samp_ref[0]
    o = lax.dot_general(wout_ref[...], sb, (((0,), (1,)), ((), ())),
                        preferred_element_type=jnp.float32)
    out_ref[0] = o + bout_ref[...] + ego_ref[0]


def _sc_gather(table, idxf, wgtf, *, bsn, dph, cq):
    qw = bsn // _NW
    nchunk = qw // cq
    mesh = plsc.VectorSubcoreMesh(core_axis_name="c", subcore_axis_name="s")

    def body(table_hbm, idx_hbm, wgt_hbm, out_hbm,
             idx_v, wgt_v, rows_v, out_v, gsem, iwsem, osem):
        wid = lax.axis_index("s") * _NC + lax.axis_index("c")
        base = wid * qw

        def start_iw(g, slot):
            q0 = base + g * cq
            pltpu.async_copy(idx_hbm.at[pl.ds(q0, cq)],
                             idx_v.at[pl.ds(slot * cq, cq)], iwsem)
            pltpu.async_copy(wgt_hbm.at[pl.ds(q0, cq)],
                             wgt_v.at[pl.ds(slot * cq, cq)], iwsem)

        def wait_iw():
            for _ in range(2):
                pltpu.make_async_copy(
                    idx_hbm.at[pl.ds(base, cq)],
                    idx_v.at[pl.ds(0, cq)], iwsem).wait()

        def start_gathers(slot):
            for q in range(cq):
                pltpu.async_copy(
                    table_hbm.at[idx_v.at[slot * cq + q]],
                    rows_v.at[pl.ds((slot * cq + q) * _K, _K)], gsem)

        def wait_gathers():
            for _ in range(cq):
                pltpu.make_async_copy(
                    table_hbm.at[idx_v.at[0]],
                    rows_v.at[pl.ds(0, _K)], gsem).wait()

        def wait_out():
            pltpu.make_async_copy(
                out_v.at[pl.ds(0, cq)],
                out_hbm.at[pl.ds(base, cq)], osem).wait()

        pltpu.sync_copy(idx_hbm.at[pl.ds(base, cq)], idx_v.at[pl.ds(0, cq)])
        pltpu.sync_copy(wgt_hbm.at[pl.ds(base, cq)], wgt_v.at[pl.ds(0, cq)])
        start_gathers(0)
        start_iw(1, 1)

        def chunk(g, carry):
            slot = g & 1
            wait_gathers()

            @pl.when(g < nchunk - 1)
            def _():
                wait_iw()
                start_gathers(1 - slot)

            @pl.when(g >= 2)
            def _():
                wait_out()

            def per_query(q, c2):
                acc = [jnp.zeros((dph,), jnp.float32) for _ in range(_HEADS)]
                for j16 in range(_K // 16):
                    wv = wgt_v[slot * cq + q, pl.ds(j16 * 16, 16)]
                    for l in range(16):
                        j = j16 * 16 + l
                        r = rows_v[(slot * cq + q) * _K + j, :]
                        hh = (j % _S) >> 2
                        acc[hh] = acc[hh] + r * wv[l]
                for hh in range(_HEADS):
                    out_v[slot * cq + q, pl.ds(hh * dph, dph)] = acc[hh]
                return c2

            lax.fori_loop(0, cq, per_query, 0)
            pltpu.async_copy(out_v.at[pl.ds(slot * cq, cq)],
                             out_hbm.at[pl.ds(base + g * cq, cq)], osem)

            @pl.when(g < nchunk - 2)
            def _():
                start_iw(g + 2, slot)

            return carry

        lax.fori_loop(0, nchunk, chunk, 0)
        wait_out()
        wait_out()

    f = pl.kernel(
        body,
        out_type=jax.ShapeDtypeStruct((bsn, _K), jnp.float32),
        mesh=mesh,
        compiler_params=pltpu.CompilerParams(use_tc_tiling_on_sc=False),
        scratch_types=[
            pltpu.VMEM((2 * cq, _K), jnp.int32),
            pltpu.VMEM((2 * cq, _K), jnp.float32),
            pltpu.VMEM((2 * cq * _K, dph), jnp.float32),
            pltpu.VMEM((2 * cq, _K), jnp.float32),
            pltpu.SemaphoreType.DMA,
            pltpu.SemaphoreType.DMA,
            pltpu.SemaphoreType.DMA,
        ],
    )
    return f(table, idxf, wgtf)


def kernel(ego_feature, protocol_feature, Wv, bv, Woff, boff, Wa, ba,
           Wout, bout):
    bs, C, H, W = ego_feature.shape
    N = H * W
    dph = C // _HEADS
    TN = 512
    CQ = 16

    ego3 = ego_feature.reshape(bs, C, N)
    proto3 = protocol_feature.reshape(bs, C, N)
    Woffx = jnp.concatenate([Woff[:, 0::2]] * _CORNERS, axis=1)
    Woffy = jnp.concatenate([Woff[:, 1::2]] * _CORNERS, axis=1)
    bo = boff.reshape(_S, 2)
    boffx = jnp.concatenate([bo[:, 0].reshape(1, _S)] * _CORNERS, axis=1)
    boffy = jnp.concatenate([bo[:, 1].reshape(1, _S)] * _CORNERS, axis=1)
    Wa4 = jnp.concatenate([Wa] * _CORNERS, axis=1)
    ba4 = jnp.concatenate([ba.reshape(1, _S)] * _CORNERS, axis=1)
    bv2 = bv.reshape(1, C)
    bout2 = bout.reshape(C, 1)
    GG = jnp.kron(jnp.eye(_S, dtype=jnp.float32),
                  jnp.ones((_POINTS, _POINTS), jnp.float32))

    nblk = N // TN
    grid = (1, nblk)

    import functools
    pre = pl.pallas_call(
        functools.partial(_pre_body, tn=TN, h_img=H, w_img=W, n_tot=N),
        grid=grid,
        in_specs=[
            pl.BlockSpec((1, C, TN), lambda b, nb: (b, 0, nb)),
            pl.BlockSpec((1, C, TN), lambda b, nb: (b, 0, nb)),
            pl.BlockSpec((C, C), lambda b, nb: (0, 0)),
            pl.BlockSpec((1, C), lambda b, nb: (0, 0)),
            pl.BlockSpec((C, _K), lambda b, nb: (0, 0)),
            pl.BlockSpec((C, _K), lambda b, nb: (0, 0)),
            pl.BlockSpec((1, _K), lambda b, nb: (0, 0)),
            pl.BlockSpec((1, _K), lambda b, nb: (0, 0)),
            pl.BlockSpec((C, _K), lambda b, nb: (0, 0)),
            pl.BlockSpec((1, _K), lambda b, nb: (0, 0)),
            pl.BlockSpec((_K, _K), lambda b, nb: (0, 0)),
        ],
        out_specs=[
            pl.BlockSpec((1, TN, C), lambda b, nb: (b, nb, 0)),
            pl.BlockSpec((1, TN, _K), lambda b, nb: (b, nb, 0)),
            pl.BlockSpec((1, TN, _K), lambda b, nb: (b, nb, 0)),
        ],
        out_shape=[
            jax.ShapeDtypeStruct((1, N, C), jnp.float32),
            jax.ShapeDtypeStruct((1, N, _K), jnp.int32),
            jax.ShapeDtypeStruct((1, N, _K), jnp.float32),
        ],
    )

    post = pl.pallas_call(
        _post_body,
        grid=grid,
        in_specs=[
            pl.BlockSpec((1, TN, C), lambda b, nb: (b, nb, 0)),
            pl.BlockSpec((1, C, TN), lambda b, nb: (b, 0, nb)),
            pl.BlockSpec((C, C), lambda b, nb: (0, 0)),
            pl.BlockSpec((C, 1), lambda b, nb: (0, 0)),
        ],
        out_specs=pl.BlockSpec((1, C, TN), lambda b, nb: (b, 0, nb)),
        out_shape=jax.ShapeDtypeStruct((1, C, N), jnp.float32),
    )

    outs = []
    for b in range(bs):
        vp, idxa, wgta = pre(ego3[b:b + 1], proto3[b:b + 1], Wv, bv2,
                             Woffx, Woffy, boffx, boffy, Wa4, ba4, GG)
        table = vp.reshape(N * _HEADS, dph)
        idxf = idxa.reshape(N, _K)
        wgtf = wgta.reshape(N, _K)
        samp = _sc_gather(table, idxf, wgtf, bsn=N, dph=dph, cq=CQ)
        outs.append(post(samp.reshape(1, N, C), ego3[b:b + 1], Wout, bout2))
    out3 = jnp.concatenate(outs, axis=0)
    return out3.reshape(bs, C, H, W)


# speedup vs baseline: 398.3219x; 1.0024x over previous
import jax
import jax.numpy as jnp
from jax import lax
from jax.experimental import pallas as pl
from jax.experimental.pallas import tpu as pltpu
from jax.experimental.pallas import tpu_sc as plsc

_HEADS = 8
_POINTS = 4
_S = _HEADS * _POINTS
_CORNERS = 4
_K = _S * _CORNERS
_NC = 2
_NS = 16
_NW = _NC * _NS


def _pre_body(ego_ref, proto_ref, wv_ref, bv_ref, wox_ref, woy_ref,
              box_ref, boy_ref, wa_ref, ba_ref, gg_ref,
              vp_ref, idx_ref, wgt_ref, *, tn, h_img, w_img, n_tot):
    b = pl.program_id(0)
    nb = pl.program_id(1)
    eb = ego_ref[0]
    vb = proto_ref[0]
    dn = (((0,), (0,)), ((), ()))

    vp = lax.dot_general(vb, wv_ref[...], dn,
                         preferred_element_type=jnp.float32) + bv_ref[...]
    vp_ref[0] = vp

    offx = lax.dot_general(eb, wox_ref[...], dn,
                           preferred_element_type=jnp.float32) + box_ref[...]
    offy = lax.dot_general(eb, woy_ref[...], dn,
                           preferred_element_type=jnp.float32) + boy_ref[...]
    logit = lax.dot_general(eb, wa_ref[...], dn,
                            preferred_element_type=jnp.float32) + ba_ref[...]
    e = jnp.exp(logit)
    denom = lax.dot_general(e, gg_ref[...], (((1,), (0,)), ((), ())),
                            preferred_element_type=jnp.float32)
    aw = e / denom

    lane = lax.broadcasted_iota(jnp.int32, (tn, _K), 1)
    cor = lane >> 5
    dx = (cor & 1).astype(jnp.float32)
    dy = (cor >> 1).astype(jnp.float32)
    head = (lane & (_S - 1)) >> 2

    n = nb * tn + lax.broadcasted_iota(jnp.int32, (tn, 1), 0)
    m = n >> 6
    i = (m * 21846) >> 16
    j = n - i * w_img
    x = j.astype(jnp.float32) + offx
    y = i.astype(jnp.float32) + offy
    x0f = jnp.floor(x)
    y0f = jnp.floor(y)
    fx1 = x - x0f
    fy1 = y - y0f
    xcf = x0f + dx
    ycf = y0f + dy
    wx = dx * fx1 + (1.0 - dx) * (1.0 - fx1)
    wy = dy * fy1 + (1.0 - dy) * (1.0 - fy1)
    valid = ((xcf >= 0.0) & (xcf <= w_img - 1) &
             (ycf >= 0.0) & (ycf <= h_img - 1))
    w = wx * wy * aw * valid.astype(jnp.float32)
    xi = jnp.clip(xcf, 0.0, w_img - 1).astype(jnp.int32)
    yi = jnp.clip(ycf, 0.0, h_img - 1).astype(jnp.int32)
    base = b * n_tot
    idx_ref[0] = ((base + yi * w_img + xi) << 3) + head
    wgt_ref[0] = w


def _post_body(samp_ref, ego_ref, wout_ref, bout_ref, out_ref):
    sb = samp_ref[0]
    o = lax.dot_general(wout_ref[...], sb, (((0,), (1,)), ((), ())),
                        preferred_element_type=jnp.float32)
    out_ref[0] = o + bout_ref[...] + ego_ref[0]


def _sc_gather(table, idxf, wgtf, *, bsn, dph, cq):
    qw = bsn // _NW
    nchunk = qw // cq
    mesh = plsc.VectorSubcoreMesh(core_axis_name="c", subcore_axis_name="s")

    def body(table_hbm, idx_hbm, wgt_hbm, out_hbm,
             idx_v, wgt_v, rows_v, out_v, gsem, iwsem, osem):
        wid = lax.axis_index("s") * _NC + lax.axis_index("c")
        base = wid * qw

        def start_iw(g, slot):
            q0 = base + g * cq
            pltpu.async_copy(idx_hbm.at[pl.ds(q0, cq)],
                             idx_v.at[pl.ds(slot * cq, cq)], iwsem)
            pltpu.async_copy(wgt_hbm.at[pl.ds(q0, cq)],
                             wgt_v.at[pl.ds(slot * cq, cq)], iwsem)

        def wait_iw():
            for _ in range(2):
                pltpu.make_async_copy(
                    idx_hbm.at[pl.ds(base, cq)],
                    idx_v.at[pl.ds(0, cq)], iwsem).wait()

        def start_gathers(slot):
            for q in range(cq):
                pltpu.async_copy(
                    table_hbm.at[idx_v.at[slot * cq + q]],
                    rows_v.at[pl.ds((slot * cq + q) * _K, _K)], gsem)

        def wait_gathers():
            pltpu.make_async_copy(
                table_hbm.at[pl.ds(0, cq * _K)],
                rows_v.at[pl.ds(0, cq * _K)], gsem).wait()

        def wait_out():
            pltpu.make_async_copy(
                out_v.at[pl.ds(0, cq)],
                out_hbm.at[pl.ds(base, cq)], osem).wait()

        pltpu.sync_copy(idx_hbm.at[pl.ds(base, cq)], idx_v.at[pl.ds(0, cq)])
        pltpu.sync_copy(wgt_hbm.at[pl.ds(base, cq)], wgt_v.at[pl.ds(0, cq)])
        start_gathers(0)
        start_iw(1, 1)

        def chunk(g, carry):
            slot = g & 1
            wait_gathers()

            @pl.when(g < nchunk - 1)
            def _():
                wait_iw()
                start_gathers(1 - slot)

            @pl.when(g >= 2)
            def _():
                wait_out()

            def per_query(q, c2):
                acc = [jnp.zeros((dph,), jnp.float32) for _ in range(_HEADS)]
                for j16 in range(_K // 16):
                    wv = wgt_v[slot * cq + q, pl.ds(j16 * 16, 16)]
                    for l in range(16):
                        j = j16 * 16 + l
                        r = rows_v[(slot * cq + q) * _K + j, :]
                        hh = (j % _S) >> 2
                        acc[hh] = acc[hh] + r * wv[l]
                for hh in range(_HEADS):
                    out_v[slot * cq + q, pl.ds(hh * dph, dph)] = acc[hh]
                return c2

            lax.fori_loop(0, cq, per_query, 0)
            pltpu.async_copy(out_v.at[pl.ds(slot * cq, cq)],
                             out_hbm.at[pl.ds(base + g * cq, cq)], osem)

            @pl.when(g < nchunk - 2)
            def _():
                start_iw(g + 2, slot)

            return carry

        lax.fori_loop(0, nchunk, chunk, 0)
        wait_out()
        wait_out()

    f = pl.kernel(
        body,
        out_type=jax.ShapeDtypeStruct((bsn, _K), jnp.float32),
        mesh=mesh,
        compiler_params=pltpu.CompilerParams(use_tc_tiling_on_sc=False),
        scratch_types=[
            pltpu.VMEM((2 * cq, _K), jnp.int32),
            pltpu.VMEM((2 * cq, _K), jnp.float32),
            pltpu.VMEM((2 * cq * _K, dph), jnp.float32),
            pltpu.VMEM((2 * cq, _K), jnp.float32),
            pltpu.SemaphoreType.DMA,
            pltpu.SemaphoreType.DMA,
            pltpu.SemaphoreType.DMA,
        ],
    )
    return f(table, idxf, wgtf)


def kernel(ego_feature, protocol_feature, Wv, bv, Woff, boff, Wa, ba,
           Wout, bout):
    bs, C, H, W = ego_feature.shape
    N = H * W
    dph = C // _HEADS
    TN = 512
    CQ = 16

    ego3 = ego_feature.reshape(bs, C, N)
    proto3 = protocol_feature.reshape(bs, C, N)
    Woffx = jnp.concatenate([Woff[:, 0::2]] * _CORNERS, axis=1)
    Woffy = jnp.concatenate([Woff[:, 1::2]] * _CORNERS, axis=1)
    bo = boff.reshape(_S, 2)
    boffx = jnp.concatenate([bo[:, 0].reshape(1, _S)] * _CORNERS, axis=1)
    boffy = jnp.concatenate([bo[:, 1].reshape(1, _S)] * _CORNERS, axis=1)
    Wa4 = jnp.concatenate([Wa] * _CORNERS, axis=1)
    ba4 = jnp.concatenate([ba.reshape(1, _S)] * _CORNERS, axis=1)
    bv2 = bv.reshape(1, C)
    bout2 = bout.reshape(C, 1)
    GG = jnp.kron(jnp.eye(_S, dtype=jnp.float32),
                  jnp.ones((_POINTS, _POINTS), jnp.float32))

    nblk = N // TN
    grid = (1, nblk)

    import functools
    pre = pl.pallas_call(
        functools.partial(_pre_body, tn=TN, h_img=H, w_img=W, n_tot=N),
        grid=grid,
        in_specs=[
            pl.BlockSpec((1, C, TN), lambda b, nb: (b, 0, nb)),
            pl.BlockSpec((1, C, TN), lambda b, nb: (b, 0, nb)),
            pl.BlockSpec((C, C), lambda b, nb: (0, 0)),
            pl.BlockSpec((1, C), lambda b, nb: (0, 0)),
            pl.BlockSpec((C, _K), lambda b, nb: (0, 0)),
            pl.BlockSpec((C, _K), lambda b, nb: (0, 0)),
            pl.BlockSpec((1, _K), lambda b, nb: (0, 0)),
            pl.BlockSpec((1, _K), lambda b, nb: (0, 0)),
            pl.BlockSpec((C, _K), lambda b, nb: (0, 0)),
            pl.BlockSpec((1, _K), lambda b, nb: (0, 0)),
            pl.BlockSpec((_K, _K), lambda b, nb: (0, 0)),
        ],
        out_specs=[
            pl.BlockSpec((1, TN, C), lambda b, nb: (b, nb, 0)),
            pl.BlockSpec((1, TN, _K), lambda b, nb: (b, nb, 0)),
            pl.BlockSpec((1, TN, _K), lambda b, nb: (b, nb, 0)),
        ],
        out_shape=[
            jax.ShapeDtypeStruct((1, N, C), jnp.float32),
            jax.ShapeDtypeStruct((1, N, _K), jnp.int32),
            jax.ShapeDtypeStruct((1, N, _K), jnp.float32),
        ],
    )

    post = pl.pallas_call(
        _post_body,
        grid=grid,
        in_specs=[
            pl.BlockSpec((1, TN, C), lambda b, nb: (b, nb, 0)),
            pl.BlockSpec((1, C, TN), lambda b, nb: (b, 0, nb)),
            pl.BlockSpec((C, C), lambda b, nb: (0, 0)),
            pl.BlockSpec((C, 1), lambda b, nb: (0, 0)),
        ],
        out_specs=pl.BlockSpec((1, C, TN), lambda b, nb: (b, 0, nb)),
        out_shape=jax.ShapeDtypeStruct((1, C, N), jnp.float32),
    )

    outs = []
    for b in range(bs):
        vp, idxa, wgta = pre(ego3[b:b + 1], proto3[b:b + 1], Wv, bv2,
                             Woffx, Woffy, boffx, boffy, Wa4, ba4, GG)
        table = vp.reshape(N * _HEADS, dph)
        idxf = idxa.reshape(N, _K)
        wgtf = wgta.reshape(N, _K)
        samp = _sc_gather(table, idxf, wgtf, bsn=N, dph=dph, cq=CQ)
        outs.append(post(samp.reshape(1, N, C), ego3[b:b + 1], Wout, bout2))
    out3 = jnp.concatenate(outs, axis=0)
    return out3.reshape(bs, C, H, W)
